# Initial kernel scaffold; baseline (speedup 1.0000x reference)
import jax
import jax.numpy as jnp
from jax.experimental import pallas as pl


def kernel(features, weight, cluster_centers, pseudo_assignment):
    raise NotImplementedError("write your pallas kernel here")


# speedup vs baseline: 1.5409x; 1.5409x over previous
import jax
import jax.numpy as jnp
from jax.experimental import pallas as pl
from jax.experimental.pallas import tpu as pltpu

_DIM = 768
_NCLUST = 8192
_TM = 1024
_TN = 1024


def _norm_body(x_ref, o_ref):
    x = x_ref[...]
    ss = jnp.sum(x * x, axis=1, keepdims=True)
    inv = 1.0 / jnp.maximum(jnp.sqrt(ss), 1e-12)
    o_ref[...] = (x * inv).astype(jnp.bfloat16)


def _normalize_bf16(x, tile):
    return pl.pallas_call(
        _norm_body,
        grid=(x.shape[0] // tile,),
        in_specs=[pl.BlockSpec((tile, _DIM), lambda i: (i, 0))],
        out_specs=pl.BlockSpec((tile, _DIM), lambda i: (i, 0)),
        out_shape=jax.ShapeDtypeStruct(x.shape, jnp.bfloat16),
    )(x)


def _main_body(nc_ref, na_ref, w_ref, t_ref, lab_ref, seg_ref, loss_ref,
               gmax_ref, gidx_ref, gseg_ref):
    j = pl.program_id(1)
    nj = pl.num_programs(1)

    p = jax.lax.dot_general(
        nc_ref[...], na_ref[...],
        dimension_numbers=(((1,), (1,)), ((), ())),
        preferred_element_type=jnp.float32,
    )

    m8 = jnp.max(p.reshape(_TN // 8, 8, _TM), axis=0)
    cmax = jnp.max(m8, axis=0, keepdims=True)

    ind = (p >= cmax).astype(jnp.float32)
    ext = jax.lax.dot_general(
        t_ref[...], ind,
        dimension_numbers=(((1,), (0,)), ((), ())),
        precision=jax.lax.Precision.HIGHEST,
        preferred_element_type=jnp.float32,
    )
    idxv = ext[0:1, :]
    segv = ext[1:2, :]

    @pl.when(j == 0)
    def _():
        gmax_ref[...] = cmax
        gidx_ref[...] = idxv
        gseg_ref[...] = segv

    @pl.when(j > 0)
    def _():
        upd = cmax > gmax_ref[...]
        gmax_ref[...] = jnp.where(upd, cmax, gmax_ref[...])
        gidx_ref[...] = jnp.where(upd, idxv, gidx_ref[...])
        gseg_ref[...] = jnp.where(upd, segv, gseg_ref[...])

    @pl.when(j == nj - 1)
    def _():
        i = pl.program_id(0)
        lab_ref[...] = gidx_ref[...].astype(jnp.int32).reshape(1, 1, _TM)
        seg_ref[...] = gseg_ref[...].astype(jnp.int32).reshape(1, 1, _TM)
        partial = jnp.sum(gmax_ref[...] * w_ref[0], keepdims=True)
        prev = jnp.where(i == 0, jnp.zeros((1, 1), jnp.float32), loss_ref[...])
        loss_ref[...] = prev - partial.reshape(1, 1) * (1.0 / 16384.0)


def kernel(features, weight, cluster_centers, pseudo_assignment):
    f = features.reshape(-1, _DIM)
    m = f.shape[0]
    nf = _normalize_bf16(f, _TM)
    nc = _normalize_bf16(cluster_centers, _TN)

    table = jnp.zeros((8, _NCLUST), jnp.float32)
    table = table.at[0].set(jnp.arange(_NCLUST, dtype=jnp.float32))
    table = table.at[1].set(pseudo_assignment.astype(jnp.float32))
    w3 = weight.reshape(m // _TM, 1, _TM)

    grid = (m // _TM, _NCLUST // _TN)
    lab3, seg3, loss2 = pl.pallas_call(
        _main_body,
        grid=grid,
        in_specs=[
            pl.BlockSpec((_TN, _DIM), lambda i, j: (j, 0)),
            pl.BlockSpec((_TM, _DIM), lambda i, j: (i, 0)),
            pl.BlockSpec((1, 1, _TM), lambda i, j: (i, 0, 0)),
            pl.BlockSpec((8, _TN), lambda i, j: (0, j)),
        ],
        out_specs=[
            pl.BlockSpec((1, 1, _TM), lambda i, j: (i, 0, 0)),
            pl.BlockSpec((1, 1, _TM), lambda i, j: (i, 0, 0)),
            pl.BlockSpec((1, 1), lambda i, j: (0, 0)),
        ],
        out_shape=[
            jax.ShapeDtypeStruct((m // _TM, 1, _TM), jnp.int32),
            jax.ShapeDtypeStruct((m // _TM, 1, _TM), jnp.int32),
            jax.ShapeDtypeStruct((1, 1), jnp.float32),
        ],
        scratch_shapes=[
            pltpu.VMEM((1, _TM), jnp.float32),
            pltpu.VMEM((1, _TM), jnp.float32),
            pltpu.VMEM((1, _TM), jnp.float32),
        ],
    )(nc, nf, w3, table)

    pseudo_segs_pred = lab3.reshape(features.shape[:-1])
    segs_pred = seg3.reshape(features.shape[:-1])
    loss = loss2.reshape(())
    return pseudo_segs_pred, segs_pred, loss
